# baseline (speedup 1.0000x reference)
import functools

import jax
import jax.numpy as jnp
from jax import lax
from jax.experimental import pallas as pl
from jax.experimental.pallas import tpu as pltpu
from jax.experimental.pallas import tpu_sc as plsc

B = 16384
D = 64
NF = 8
EN = 12
NROWS = NF * EN
NP = NF // 2
EN2 = EN * EN
NPROWS = NP * EN2
L = 16
NQ = D // L


def _prep_body(mu_ref, std_ref, act_ref, spp_ref, mudp_ref):
    spt = 0.01 * jnp.log(1.0 + jnp.exp(std_ref[...]))
    mud = lax.dot_general(
        act_ref[...], mu_ref[...], (((1,), (1,)), ((), ())),
        preferred_element_type=jnp.float32)
    for p in range(NP):
        a = spt[2 * p * EN:(2 * p + 1) * EN]
        b = spt[(2 * p + 1) * EN:(2 * p + 2) * EN]
        pair = a[:, None, :] + b[None, :, :]
        spp_ref[pl.ds(p * EN2, EN2)] = pair.reshape(EN2, D)
        ma = mud[:, 2 * p * EN:(2 * p + 1) * EN]
        mb = mud[:, (2 * p + 1) * EN:(2 * p + 2) * EN]
        mpair = ma[:, :, None] + mb[:, None, :]
        mudp_ref[:, pl.ds(p * EN2, EN2)] = mpair.reshape(2, EN2)


def _prep(mu_flat, std_flat, act):
    return pl.pallas_call(
        _prep_body,
        out_shape=(
            jax.ShapeDtypeStruct((NPROWS, D), jnp.float32),
            jax.ShapeDtypeStruct((2, NPROWS), jnp.float32),
        ),
    )(mu_flat, std_flat, act)


@functools.cache
def _build_sc():
    info = plsc.get_sparse_core_info()
    nc, ns = info.num_cores, info.num_subcores
    nw = nc * ns
    bw = B // nw
    nt = bw // L
    mesh = plsc.VectorSubcoreMesh(core_axis_name="c", subcore_axis_name="s")

    @functools.partial(
        pl.kernel,
        out_type=jax.ShapeDtypeStruct((2 * B,), jnp.float32),
        mesh=mesh,
        compiler_params=pltpu.CompilerParams(needs_layout_passes=False),
        scratch_types=[
            pltpu.VMEM((NF, bw), jnp.int32),
            pltpu.VMEM((bw * D,), jnp.float32),
            pltpu.VMEM((NPROWS * D,), jnp.float32),
            pltpu.VMEM((NPROWS,), jnp.float32),
            pltpu.VMEM((NPROWS,), jnp.float32),
            pltpu.VMEM((2, D), jnp.float32),
            pltpu.VMEM((NP * bw,), jnp.int32),
            pltpu.VMEM((bw * L,), jnp.float32),
            pltpu.VMEM((bw * L,), jnp.float32),
            pltpu.VMEM((2 * bw,), jnp.float32),
            pltpu.SemaphoreType.DMA,
        ],
    )
    def fm_sc(i0_hbm, i1_hbm, i2_hbm, i3_hbm, i4_hbm, i5_hbm, i6_hbm, i7_hbm,
              rand_hbm, spp_hbm, mudp_hbm, act_hbm,
              o_hbm,
              idx_v, v_v, spp_v, mudp0_v, mudp1_v, act_v, pidx_v,
              p0_v, p1_v, o_v, sem):
        wid = lax.axis_index("s") * nc + lax.axis_index("c")
        base = wid * bw
        idx_hbms = [i0_hbm, i1_hbm, i2_hbm, i3_hbm, i4_hbm, i5_hbm, i6_hbm,
                    i7_hbm]
        copies = [
            pltpu.async_copy(idx_hbms[i].at[pl.ds(base, bw)], idx_v.at[i],
                             sem)
            for i in range(NF)
        ]
        copies += [
            pltpu.async_copy(rand_hbm.at[pl.ds(base * D, bw * D)], v_v, sem),
            pltpu.async_copy(spp_hbm, spp_v, sem),
            pltpu.async_copy(mudp_hbm.at[0], mudp0_v, sem),
            pltpu.async_copy(mudp_hbm.at[1], mudp1_v, sem),
            pltpu.async_copy(act_hbm, act_v, sem),
        ]
        for c in copies:
            c.wait()

        a0 = [act_v[0, pl.ds(q * L, L)] for q in range(NQ)]
        a1 = [act_v[1, pl.ds(q * L, L)] for q in range(NQ)]

        def grp_body(t, carry):
            ivs = [idx_v[i, pl.ds(t * L, L)] for i in range(NF)]
            pvs = [ivs[2 * i] * EN + ivs[2 * i + 1] + i * EN2
                   for i in range(NP)]
            for i in range(NP):
                pidx_v[pl.ds(i * bw + t * L, L)] = pvs[i]
            rvs = [pvs[i] * D for i in range(NP)]
            for k in range(L):
                b = t * L + k
                f = [rvs[i][k] for i in range(NP)]
                p0 = None
                p1 = None
                for q in range(NQ):
                    g = spp_v[pl.ds(f[0] + q * L, L)]
                    for i in range(1, NP):
                        g = g + spp_v[pl.ds(f[i] + q * L, L)]
                    m = g * v_v[pl.ds(b * D + q * L, L)]
                    t0 = m * a0[q]
                    t1 = m * a1[q]
                    p0 = t0 if q == 0 else p0 + t0
                    p1 = t1 if q == 0 else p1 + t1
                p0_v[pl.ds(b * L, L)] = p0
                p1_v[pl.ds(b * L, L)] = p1
            return carry

        lax.fori_loop(0, nt, grp_body, 0)

        iota = lax.iota(jnp.int32, L)

        def red_body(t, carry):
            rowbase = (t * L + iota) * L
            acc0 = plsc.load_gather(p0_v, [rowbase])
            acc1 = plsc.load_gather(p1_v, [rowbase])
            for j in range(1, L):
                acc0 = acc0 + plsc.load_gather(p0_v, [rowbase + j])
                acc1 = acc1 + plsc.load_gather(p1_v, [rowbase + j])
            for i in range(NP):
                fi = pidx_v[pl.ds(i * bw + t * L, L)]
                acc0 = acc0 + plsc.load_gather(mudp0_v, [fi])
                acc1 = acc1 + plsc.load_gather(mudp1_v, [fi])
            two_row = 2 * (t * L + iota)
            plsc.store_scatter(o_v, [two_row], acc0)
            plsc.store_scatter(o_v, [two_row + 1], acc1)
            return carry

        lax.fori_loop(0, nt, red_body, 0)

        pltpu.sync_copy(o_v, o_hbm.at[pl.ds(2 * base, 2 * bw)])

    return fm_sc


def kernel(workclass, education, marital_status, occupation, relationship,
           race, sex, native_country, label, mean_tables, std_tables,
           action_table, rand_array):
    mu_flat = mean_tables.reshape(NROWS, D)
    std_flat = std_tables.reshape(NROWS, D)
    spp, mudp = _prep(mu_flat, std_flat, action_table)
    o = _build_sc()(workclass, education, marital_status, occupation,
                    relationship, race, sex, native_country,
                    rand_array, spp.reshape(NPROWS * D), mudp, action_table)
    return o.reshape(B, 2)


# speedup vs baseline: 1.3088x; 1.3088x over previous
import functools

import jax
import jax.numpy as jnp
from jax import lax
from jax.experimental import pallas as pl
from jax.experimental.pallas import tpu as pltpu
from jax.experimental.pallas import tpu_sc as plsc

B = 16384
D = 64
NF = 8
EN = 12
NROWS = NF * EN
NP = NF // 2
EN2 = EN * EN
NPROWS = NP * EN2
L = 16
NQ = D // L


def _prep_body(mu_ref, std_ref, act_ref, spp_ref, mudp_ref):
    spt = 0.01 * jnp.log(1.0 + jnp.exp(std_ref[...]))
    mud = lax.dot_general(
        act_ref[...], mu_ref[...], (((1,), (1,)), ((), ())),
        preferred_element_type=jnp.float32)
    for p in range(NP):
        a = spt[2 * p * EN:(2 * p + 1) * EN]
        b = spt[(2 * p + 1) * EN:(2 * p + 2) * EN]
        pair = a[:, None, :] + b[None, :, :]
        spp_ref[pl.ds(p * EN2, EN2)] = pair.reshape(EN2, D)
        ma = mud[:, 2 * p * EN:(2 * p + 1) * EN]
        mb = mud[:, (2 * p + 1) * EN:(2 * p + 2) * EN]
        mpair = ma[:, :, None] + mb[:, None, :]
        mudp_ref[:, pl.ds(p * EN2, EN2)] = mpair.reshape(2, EN2)


def _prep(mu_flat, std_flat, act):
    return pl.pallas_call(
        _prep_body,
        out_shape=(
            jax.ShapeDtypeStruct((NPROWS, D), jnp.float32),
            jax.ShapeDtypeStruct((2, NPROWS), jnp.float32),
        ),
    )(mu_flat, std_flat, act)


@functools.cache
def _build_sc():
    info = plsc.get_sparse_core_info()
    nc, ns = info.num_cores, info.num_subcores
    nw = nc * ns
    bw = B // nw
    nt = bw // L
    mesh = plsc.VectorSubcoreMesh(core_axis_name="c", subcore_axis_name="s")

    @functools.partial(
        pl.kernel,
        out_type=(
            jax.ShapeDtypeStruct((B,), jnp.float32),
            jax.ShapeDtypeStruct((B,), jnp.float32),
        ),
        mesh=mesh,
        compiler_params=pltpu.CompilerParams(needs_layout_passes=False),
        scratch_types=[
            pltpu.VMEM((NF, bw), jnp.int32),
            pltpu.VMEM((bw * D,), jnp.float32),
            pltpu.VMEM((NPROWS * D,), jnp.float32),
            pltpu.VMEM((NPROWS,), jnp.float32),
            pltpu.VMEM((NPROWS,), jnp.float32),
            pltpu.VMEM((2, D), jnp.float32),
            pltpu.VMEM((NP * bw,), jnp.int32),
            pltpu.VMEM((bw * L,), jnp.float32),
            pltpu.VMEM((bw * L,), jnp.float32),
            pltpu.VMEM((bw,), jnp.float32),
            pltpu.VMEM((bw,), jnp.float32),
            pltpu.SemaphoreType.DMA,
        ],
    )
    def fm_sc(i0_hbm, i1_hbm, i2_hbm, i3_hbm, i4_hbm, i5_hbm, i6_hbm, i7_hbm,
              rand_hbm, spp_hbm, mudp_hbm, act_hbm,
              o0_hbm, o1_hbm,
              idx_v, v_v, spp_v, mudp0_v, mudp1_v, act_v, pidx_v,
              p0_v, p1_v, o0_v, o1_v, sem):
        wid = lax.axis_index("s") * nc + lax.axis_index("c")
        base = wid * bw
        idx_hbms = [i0_hbm, i1_hbm, i2_hbm, i3_hbm, i4_hbm, i5_hbm, i6_hbm,
                    i7_hbm]
        copies = [
            pltpu.async_copy(idx_hbms[i].at[pl.ds(base, bw)], idx_v.at[i],
                             sem)
            for i in range(NF)
        ]
        copies += [
            pltpu.async_copy(rand_hbm.at[pl.ds(base * D, bw * D)], v_v, sem),
            pltpu.async_copy(spp_hbm, spp_v, sem),
            pltpu.async_copy(mudp_hbm.at[0], mudp0_v, sem),
            pltpu.async_copy(mudp_hbm.at[1], mudp1_v, sem),
            pltpu.async_copy(act_hbm, act_v, sem),
        ]
        for c in copies:
            c.wait()

        a0 = [act_v[0, pl.ds(q * L, L)] for q in range(NQ)]
        a1 = [act_v[1, pl.ds(q * L, L)] for q in range(NQ)]

        def grp_body(t, carry):
            ivs = [idx_v[i, pl.ds(t * L, L)] for i in range(NF)]
            pvs = [ivs[2 * i] * EN + ivs[2 * i + 1] + i * EN2
                   for i in range(NP)]
            for i in range(NP):
                pidx_v[pl.ds(i * bw + t * L, L)] = pvs[i]
            rvs = [pvs[i] * D for i in range(NP)]
            for k in range(L):
                b = t * L + k
                f = [rvs[i][k] for i in range(NP)]
                p0 = None
                p1 = None
                for q in range(NQ):
                    g = spp_v[pl.ds(f[0] + q * L, L)]
                    for i in range(1, NP):
                        g = g + spp_v[pl.ds(f[i] + q * L, L)]
                    m = g * v_v[pl.ds(b * D + q * L, L)]
                    t0 = m * a0[q]
                    t1 = m * a1[q]
                    p0 = t0 if q == 0 else p0 + t0
                    p1 = t1 if q == 0 else p1 + t1
                p0_v[pl.ds(b * L, L)] = p0
                p1_v[pl.ds(b * L, L)] = p1
            return carry

        lax.fori_loop(0, nt, grp_body, 0)

        iota = lax.iota(jnp.int32, L)

        def red_body(t, carry):
            rowbase = (t * L + iota) * L
            acc0 = plsc.load_gather(p0_v, [rowbase])
            acc1 = plsc.load_gather(p1_v, [rowbase])
            for j in range(1, L):
                acc0 = acc0 + plsc.load_gather(p0_v, [rowbase + j])
                acc1 = acc1 + plsc.load_gather(p1_v, [rowbase + j])
            for i in range(NP):
                fi = pidx_v[pl.ds(i * bw + t * L, L)]
                acc0 = acc0 + plsc.load_gather(mudp0_v, [fi])
                acc1 = acc1 + plsc.load_gather(mudp1_v, [fi])
            o0_v[pl.ds(t * L, L)] = acc0
            o1_v[pl.ds(t * L, L)] = acc1
            return carry

        lax.fori_loop(0, nt, red_body, 0)

        pltpu.sync_copy(o0_v, o0_hbm.at[pl.ds(base, bw)])
        pltpu.sync_copy(o1_v, o1_hbm.at[pl.ds(base, bw)])

    return fm_sc


def kernel(workclass, education, marital_status, occupation, relationship,
           race, sex, native_country, label, mean_tables, std_tables,
           action_table, rand_array):
    mu_flat = mean_tables.reshape(NROWS, D)
    std_flat = std_tables.reshape(NROWS, D)
    spp, mudp = _prep(mu_flat, std_flat, action_table)
    o0, o1 = _build_sc()(workclass, education, marital_status, occupation,
                         relationship, race, sex, native_country,
                         rand_array, spp.reshape(NPROWS * D), mudp,
                         action_table)
    return jnp.stack([o0, o1], axis=1)
